# baseline (speedup 1.0000x reference)
import jax
import jax.numpy as jnp
from jax import lax
from jax.experimental import pallas as pl
from jax.experimental.pallas import tpu as pltpu

_BM = 200
_NBUF = 4


def _gcn_body(w_ref, b_ref, x_hbm, adj_hbm, o_hbm,
              x_ref, h_ref, bufs, obuf, x_sem, in_sems, out_sems):
    n = adj_hbm.shape[0]
    nblk = n // _BM

    def in_copy(blk, slot):
        return pltpu.make_async_copy(
            adj_hbm.at[pl.ds(blk * _BM, _BM), :], bufs.at[slot],
            in_sems.at[slot])

    def out_copy(blk, slot):
        return pltpu.make_async_copy(
            obuf.at[slot], o_hbm.at[pl.ds(blk * _BM, _BM), :],
            out_sems.at[slot])

    x_copy = pltpu.make_async_copy(x_hbm, x_ref, x_sem)
    x_copy.start()
    for s in range(_NBUF - 1):
        in_copy(s, s).start()
    x_copy.wait()

    h_ref[...] = jnp.dot(
        x_ref[...].astype(jnp.bfloat16),
        w_ref[...].astype(jnp.bfloat16),
        preferred_element_type=jnp.float32,
    ).astype(jnp.bfloat16)

    def step(i, carry):
        slot = lax.rem(i, _NBUF)
        in_copy(i, slot).wait()

        @pl.when(i + _NBUF - 1 < nblk)
        def _():
            in_copy(i + _NBUF - 1, lax.rem(i + _NBUF - 1, _NBUF)).start()

        oslot = lax.rem(i, 2)

        @pl.when(i >= 2)
        def _():
            out_copy(i - 2, oslot).wait()

        obuf[oslot] = jnp.dot(
            bufs[slot].astype(jnp.bfloat16), h_ref[...],
            preferred_element_type=jnp.float32,
        ) + b_ref[...]
        out_copy(i, oslot).start()
        return carry

    lax.fori_loop(0, nblk, step, 0, unroll=2)
    out_copy(nblk - 2, (nblk - 2) % 2).wait()
    out_copy(nblk - 1, (nblk - 1) % 2).wait()


def kernel(input, adj, W, b):
    n, in_dim = input.shape
    out_dim = W.shape[1]
    b2 = b.reshape(1, out_dim)
    out = pl.pallas_call(
        _gcn_body,
        in_specs=[
            pl.BlockSpec((in_dim, out_dim), lambda: (0, 0)),
            pl.BlockSpec((1, out_dim), lambda: (0, 0)),
            pl.BlockSpec(memory_space=pltpu.HBM),
            pl.BlockSpec(memory_space=pltpu.HBM),
        ],
        out_specs=pl.BlockSpec(memory_space=pltpu.HBM),
        out_shape=jax.ShapeDtypeStruct((n, out_dim), jnp.float32),
        scratch_shapes=[
            pltpu.VMEM((n, in_dim), jnp.float32),
            pltpu.VMEM((n, out_dim), jnp.bfloat16),
            pltpu.VMEM((_NBUF, _BM, n), jnp.float32),
            pltpu.VMEM((2, _BM, out_dim), jnp.float32),
            pltpu.SemaphoreType.DMA,
            pltpu.SemaphoreType.DMA((_NBUF,)),
            pltpu.SemaphoreType.DMA((2,)),
        ],
    )(W, b2, input, adj)
    return out
